# Initial kernel scaffold; baseline (speedup 1.0000x reference)
import functools

import jax
import jax.numpy as jnp
from jax.experimental import pallas as pl
from jax.experimental.pallas import tpu as pltpu


def _pick(n, cands):
    for c in cands:
        if n % c == 0:
            return c
    return n


_DN = (((1,), (1,)), ((), ()))


def _fts_body(s1_ref, s2_ref, w_ref, o_ref, *, Dh):
    w = w_ref[:]
    o_ref[:, :Dh] = jax.lax.dot_general(
        s1_ref[:], w, _DN, preferred_element_type=jnp.float32)
    o_ref[:, Dh:] = jax.lax.dot_general(
        s2_ref[:], w, _DN, preferred_element_type=jnp.float32)


def _spmm_body(adj_ref, fts_ref, mskT_ref, bias_ref, a_ref, out_ref,
               acc_ref, *, Dh):
    i = pl.program_id(0)
    h = jax.lax.dot_general(
        adj_ref[:], fts_ref[:], (((1,), (0,)), ((), ())),
        preferred_element_type=jnp.float32)
    h = h + bias_ref[:]
    a = a_ref[0, 0]
    h = jnp.where(h >= 0, h, a * h)
    out_ref[:] = h
    part = jnp.sum(h[:, :Dh] * mskT_ref[:], axis=0, keepdims=True)

    @pl.when(i == 0)
    def _():
        acc_ref[:] = part

    @pl.when(i != 0)
    def _():
        acc_ref[:] = acc_ref[:] + part


def _v_body(hsum_ref, msk_ref, bw_ref, v_ref):
    total = jnp.sum(msk_ref[:])
    c = jax.nn.sigmoid(hsum_ref[:] / total)
    v_ref[:] = jax.lax.dot_general(
        c, bw_ref[:], _DN, preferred_element_type=jnp.float32)


def _sc_body(h_ref, v_ref, sb1_ref, sb2_ref, b_ref, sc1_ref, sc2_ref, *, Dh):
    v = v_ref[:]
    b = b_ref[0, 0]
    sc1_ref[:] = jax.lax.dot_general(
        h_ref[:, :Dh], v, _DN, preferred_element_type=jnp.float32) + b + sb1_ref[:]
    sc2_ref[:] = jax.lax.dot_general(
        h_ref[:, Dh:], v, _DN, preferred_element_type=jnp.float32) + b + sb2_ref[:]


def kernel(seq1, seq2, adj, sparse, msk, samp_bias1, samp_bias2,
           W, prelu_a, gcn_bias, bil_w, bil_b):
    B, N, Din = seq1.shape
    Dh = W.shape[0]
    s1 = seq1[0]
    s2 = seq2[0]
    A = adj[0]
    mskT = msk.reshape(N, 1)
    sb1 = samp_bias1.reshape(N, 1)
    sb2 = samp_bias2.reshape(N, 1)
    a11 = prelu_a.reshape(1, 1)
    b11 = bil_b.reshape(1, 1)
    bias_cat = jnp.concatenate([gcn_bias, gcn_bias]).reshape(1, 2 * Dh)
    bw = bil_w[0]

    Rf = _pick(N, (2000, 1000, 500, 8))
    fts = pl.pallas_call(
        functools.partial(_fts_body, Dh=Dh),
        grid=(N // Rf,),
        in_specs=[
            pl.BlockSpec((Rf, Din), lambda i: (i, 0)),
            pl.BlockSpec((Rf, Din), lambda i: (i, 0)),
            pl.BlockSpec((Dh, Din), lambda i: (0, 0)),
        ],
        out_specs=pl.BlockSpec((Rf, 2 * Dh), lambda i: (i, 0)),
        out_shape=jax.ShapeDtypeStruct((N, 2 * Dh), jnp.float32),
    )(s1, s2, W)

    R = _pick(N, (400, 200, 8))
    h_cat, hsum = pl.pallas_call(
        functools.partial(_spmm_body, Dh=Dh),
        grid=(N // R,),
        in_specs=[
            pl.BlockSpec((R, N), lambda i: (i, 0)),
            pl.BlockSpec((N, 2 * Dh), lambda i: (0, 0)),
            pl.BlockSpec((R, 1), lambda i: (i, 0)),
            pl.BlockSpec((1, 2 * Dh), lambda i: (0, 0)),
            pl.BlockSpec(memory_space=pltpu.SMEM),
        ],
        out_specs=[
            pl.BlockSpec((R, 2 * Dh), lambda i: (i, 0)),
            pl.BlockSpec((1, Dh), lambda i: (0, 0)),
        ],
        out_shape=[
            jax.ShapeDtypeStruct((N, 2 * Dh), jnp.float32),
            jax.ShapeDtypeStruct((1, Dh), jnp.float32),
        ],
    )(A, fts, mskT, bias_cat, a11)

    v = pl.pallas_call(
        _v_body,
        in_specs=[
            pl.BlockSpec((1, Dh), lambda: (0, 0)),
            pl.BlockSpec((1, N), lambda: (0, 0)),
            pl.BlockSpec((Dh, Dh), lambda: (0, 0)),
        ],
        out_specs=pl.BlockSpec((1, Dh), lambda: (0, 0)),
        out_shape=jax.ShapeDtypeStruct((1, Dh), jnp.float32),
    )(hsum, msk, bw)

    Rs = _pick(N, (2000, 1000, 500, 8))
    sc1, sc2 = pl.pallas_call(
        functools.partial(_sc_body, Dh=Dh),
        grid=(N // Rs,),
        in_specs=[
            pl.BlockSpec((Rs, 2 * Dh), lambda i: (i, 0)),
            pl.BlockSpec((1, Dh), lambda i: (0, 0)),
            pl.BlockSpec((Rs, 1), lambda i: (i, 0)),
            pl.BlockSpec((Rs, 1), lambda i: (i, 0)),
            pl.BlockSpec(memory_space=pltpu.SMEM),
        ],
        out_specs=[
            pl.BlockSpec((Rs, 1), lambda i: (i, 0)),
            pl.BlockSpec((Rs, 1), lambda i: (i, 0)),
        ],
        out_shape=[
            jax.ShapeDtypeStruct((N, 1), jnp.float32),
            jax.ShapeDtypeStruct((N, 1), jnp.float32),
        ],
    )(h_cat, v, sb1, sb2, b11)

    return jnp.concatenate([sc1.reshape(1, N), sc2.reshape(1, N)], axis=1)


# speedup vs baseline: 1.5464x; 1.5464x over previous
import functools

import jax
import jax.numpy as jnp
from jax.experimental import pallas as pl
from jax.experimental.pallas import tpu as pltpu


def _pick(n, cands):
    for c in cands:
        if n % c == 0:
            return c
    return n


_DN = (((1,), (1,)), ((), ()))


def _fts_body(s1_ref, s2_ref, w_ref, o_ref, *, Dh):
    w = w_ref[:]
    o_ref[:, :Dh] = jax.lax.dot_general(
        s1_ref[:], w, _DN, preferred_element_type=jnp.float32)
    o_ref[:, Dh:] = jax.lax.dot_general(
        s2_ref[:], w, _DN, preferred_element_type=jnp.float32)


def _spmm_body(adj_ref, fts_ref, mskT_ref, bias_ref, a_ref, out_ref,
               acc_ref, *, Dh):
    i = pl.program_id(0)
    h = jax.lax.dot_general(
        adj_ref[:], fts_ref[:], (((1,), (0,)), ((), ())),
        preferred_element_type=jnp.float32)
    h = h + bias_ref[:]
    a = a_ref[0, 0]
    h = jnp.where(h >= 0, h, a * h)
    out_ref[:] = h
    part = jnp.sum(h[:, :Dh] * mskT_ref[:], axis=0, keepdims=True)

    @pl.when(i == 0)
    def _():
        acc_ref[:] = part

    @pl.when(i != 0)
    def _():
        acc_ref[:] = acc_ref[:] + part


def _v_body(hsum_ref, msk_ref, bw_ref, v_ref):
    total = jnp.sum(msk_ref[:])
    c = jax.nn.sigmoid(hsum_ref[:] / total)
    v_ref[:] = jax.lax.dot_general(
        c, bw_ref[:], _DN, preferred_element_type=jnp.float32)


def _sc_body(h_ref, m_ref, sb_ref, b_ref, out_ref):
    b = b_ref[0, 0]
    res = jax.lax.dot_general(
        h_ref[:], m_ref[:], (((1,), (0,)), ((), ())),
        preferred_element_type=jnp.float32)
    out_ref[:] = res[:, :2] + sb_ref[:] + b


def kernel(seq1, seq2, adj, sparse, msk, samp_bias1, samp_bias2,
           W, prelu_a, gcn_bias, bil_w, bil_b):
    B, N, Din = seq1.shape
    Dh = W.shape[0]
    s1 = seq1[0]
    s2 = seq2[0]
    A = adj[0]
    mskT = msk.reshape(N, 1)
    sb1 = samp_bias1.reshape(N, 1)
    sb2 = samp_bias2.reshape(N, 1)
    a11 = prelu_a.reshape(1, 1)
    b11 = bil_b.reshape(1, 1)
    bias_cat = jnp.concatenate([gcn_bias, gcn_bias]).reshape(1, 2 * Dh)
    bw = bil_w[0]

    Rf = _pick(N, (2000, 1000, 500, 8))
    fts = pl.pallas_call(
        functools.partial(_fts_body, Dh=Dh),
        grid=(N // Rf,),
        in_specs=[
            pl.BlockSpec((Rf, Din), lambda i: (i, 0)),
            pl.BlockSpec((Rf, Din), lambda i: (i, 0)),
            pl.BlockSpec((Dh, Din), lambda i: (0, 0)),
        ],
        out_specs=pl.BlockSpec((Rf, 2 * Dh), lambda i: (i, 0)),
        out_shape=jax.ShapeDtypeStruct((N, 2 * Dh), jnp.float32),
    )(s1, s2, W)

    R = _pick(N, (400, 200, 8))
    h_cat, hsum = pl.pallas_call(
        functools.partial(_spmm_body, Dh=Dh),
        grid=(N // R,),
        in_specs=[
            pl.BlockSpec((R, N), lambda i: (i, 0)),
            pl.BlockSpec((N, 2 * Dh), lambda i: (0, 0)),
            pl.BlockSpec((R, 1), lambda i: (i, 0)),
            pl.BlockSpec((1, 2 * Dh), lambda i: (0, 0)),
            pl.BlockSpec(memory_space=pltpu.SMEM),
        ],
        out_specs=[
            pl.BlockSpec((R, 2 * Dh), lambda i: (i, 0)),
            pl.BlockSpec((1, Dh), lambda i: (0, 0)),
        ],
        out_shape=[
            jax.ShapeDtypeStruct((N, 2 * Dh), jnp.float32),
            jax.ShapeDtypeStruct((1, Dh), jnp.float32),
        ],
    )(A, fts, mskT, bias_cat, a11)

    v = pl.pallas_call(
        _v_body,
        in_specs=[
            pl.BlockSpec((1, Dh), lambda: (0, 0)),
            pl.BlockSpec((1, N), lambda: (0, 0)),
            pl.BlockSpec((Dh, Dh), lambda: (0, 0)),
        ],
        out_specs=pl.BlockSpec((1, Dh), lambda: (0, 0)),
        out_shape=jax.ShapeDtypeStruct((1, Dh), jnp.float32),
    )(hsum, msk, bw)

    vflat = v.reshape(Dh)
    zcol = jnp.zeros((Dh,), jnp.float32)
    M = jnp.stack(
        [jnp.concatenate([vflat, zcol]), jnp.concatenate([zcol, vflat])],
        axis=1)
    M = jnp.pad(M, ((0, 0), (0, 126)))
    sbcat = jnp.concatenate([sb1, sb2], axis=1)

    Rs = _pick(N, (2000, 1000, 500, 8))
    sc = pl.pallas_call(
        _sc_body,
        grid=(N // Rs,),
        in_specs=[
            pl.BlockSpec((Rs, 2 * Dh), lambda i: (i, 0)),
            pl.BlockSpec((2 * Dh, 128), lambda i: (0, 0)),
            pl.BlockSpec((Rs, 2), lambda i: (i, 0)),
            pl.BlockSpec(memory_space=pltpu.SMEM),
        ],
        out_specs=pl.BlockSpec((Rs, 2), lambda i: (i, 0)),
        out_shape=jax.ShapeDtypeStruct((N, 2), jnp.float32),
    )(h_cat, M, sbcat, b11)

    return jnp.concatenate([sc[:, 0].reshape(1, N), sc[:, 1].reshape(1, N)],
                           axis=1)


# speedup vs baseline: 1.5674x; 1.0136x over previous
import functools

import jax
import jax.numpy as jnp
from jax.experimental import pallas as pl
from jax.experimental.pallas import tpu as pltpu


def _pick(n, cands):
    for c in cands:
        if n % c == 0:
            return c
    return n


_DN = (((1,), (1,)), ((), ()))


def _fts_body(s1_ref, s2_ref, w_ref, o_ref, *, Dh):
    w = w_ref[:]
    o_ref[:, :Dh] = jax.lax.dot_general(
        s1_ref[:], w, _DN, preferred_element_type=jnp.float32
    ).astype(jnp.bfloat16)
    o_ref[:, Dh:] = jax.lax.dot_general(
        s2_ref[:], w, _DN, preferred_element_type=jnp.float32
    ).astype(jnp.bfloat16)


def _spmm_body(adj_ref, fts_ref, mskT_ref, bias_ref, a_ref, out_ref,
               acc_ref, *, Dh):
    i = pl.program_id(0)
    h = jax.lax.dot_general(
        adj_ref[:].astype(jnp.bfloat16), fts_ref[:],
        (((1,), (0,)), ((), ())),
        preferred_element_type=jnp.float32)
    h = h + bias_ref[:]
    a = a_ref[0, 0]
    h = jnp.where(h >= 0, h, a * h)
    out_ref[:] = h
    part = jnp.sum(h[:, :Dh] * mskT_ref[:], axis=0, keepdims=True)

    @pl.when(i == 0)
    def _():
        acc_ref[:] = part

    @pl.when(i != 0)
    def _():
        acc_ref[:] = acc_ref[:] + part


def _v_body(hsum_ref, msk_ref, bw_ref, v_ref):
    total = jnp.sum(msk_ref[:])
    c = jax.nn.sigmoid(hsum_ref[:] / total)
    v_ref[:] = jax.lax.dot_general(
        c, bw_ref[:], _DN, preferred_element_type=jnp.float32)


def _sc_body(h_ref, m_ref, sb_ref, b_ref, out_ref):
    b = b_ref[0, 0]
    res = jax.lax.dot_general(
        h_ref[:], m_ref[:], (((1,), (0,)), ((), ())),
        preferred_element_type=jnp.float32)
    out_ref[:] = res[:, :2] + sb_ref[:] + b


def kernel(seq1, seq2, adj, sparse, msk, samp_bias1, samp_bias2,
           W, prelu_a, gcn_bias, bil_w, bil_b):
    B, N, Din = seq1.shape
    Dh = W.shape[0]
    s1 = seq1[0]
    s2 = seq2[0]
    A = adj[0]
    mskT = msk.reshape(N, 1)
    sb1 = samp_bias1.reshape(N, 1)
    sb2 = samp_bias2.reshape(N, 1)
    a11 = prelu_a.reshape(1, 1)
    b11 = bil_b.reshape(1, 1)
    bias_cat = jnp.concatenate([gcn_bias, gcn_bias]).reshape(1, 2 * Dh)
    bw = bil_w[0]

    Rf = _pick(N, (2000, 1000, 500, 8))
    fts = pl.pallas_call(
        functools.partial(_fts_body, Dh=Dh),
        grid=(N // Rf,),
        in_specs=[
            pl.BlockSpec((Rf, Din), lambda i: (i, 0)),
            pl.BlockSpec((Rf, Din), lambda i: (i, 0)),
            pl.BlockSpec((Dh, Din), lambda i: (0, 0)),
        ],
        out_specs=pl.BlockSpec((Rf, 2 * Dh), lambda i: (i, 0)),
        out_shape=jax.ShapeDtypeStruct((N, 2 * Dh), jnp.bfloat16),
    )(s1, s2, W)

    R = _pick(N, (400, 200, 8))
    h_cat, hsum = pl.pallas_call(
        functools.partial(_spmm_body, Dh=Dh),
        grid=(N // R,),
        in_specs=[
            pl.BlockSpec((R, N), lambda i: (i, 0)),
            pl.BlockSpec((N, 2 * Dh), lambda i: (0, 0)),
            pl.BlockSpec((R, 1), lambda i: (i, 0)),
            pl.BlockSpec((1, 2 * Dh), lambda i: (0, 0)),
            pl.BlockSpec(memory_space=pltpu.SMEM),
        ],
        out_specs=[
            pl.BlockSpec((R, 2 * Dh), lambda i: (i, 0)),
            pl.BlockSpec((1, Dh), lambda i: (0, 0)),
        ],
        out_shape=[
            jax.ShapeDtypeStruct((N, 2 * Dh), jnp.float32),
            jax.ShapeDtypeStruct((1, Dh), jnp.float32),
        ],
    )(A, fts, mskT, bias_cat, a11)

    v = pl.pallas_call(
        _v_body,
        in_specs=[
            pl.BlockSpec((1, Dh), lambda: (0, 0)),
            pl.BlockSpec((1, N), lambda: (0, 0)),
            pl.BlockSpec((Dh, Dh), lambda: (0, 0)),
        ],
        out_specs=pl.BlockSpec((1, Dh), lambda: (0, 0)),
        out_shape=jax.ShapeDtypeStruct((1, Dh), jnp.float32),
    )(hsum, msk, bw)

    vflat = v.reshape(Dh)
    zcol = jnp.zeros((Dh,), jnp.float32)
    M = jnp.stack(
        [jnp.concatenate([vflat, zcol]), jnp.concatenate([zcol, vflat])],
        axis=1)
    M = jnp.pad(M, ((0, 0), (0, 126)))
    sbcat = jnp.concatenate([sb1, sb2], axis=1)

    Rs = _pick(N, (2000, 1000, 500, 8))
    sc = pl.pallas_call(
        _sc_body,
        grid=(N // Rs,),
        in_specs=[
            pl.BlockSpec((Rs, 2 * Dh), lambda i: (i, 0)),
            pl.BlockSpec((2 * Dh, 128), lambda i: (0, 0)),
            pl.BlockSpec((Rs, 2), lambda i: (i, 0)),
            pl.BlockSpec(memory_space=pltpu.SMEM),
        ],
        out_specs=pl.BlockSpec((Rs, 2), lambda i: (i, 0)),
        out_shape=jax.ShapeDtypeStruct((N, 2), jnp.float32),
    )(h_cat, M, sbcat, b11)

    return jnp.concatenate([sc[:, 0].reshape(1, N), sc[:, 1].reshape(1, N)],
                           axis=1)


# speedup vs baseline: 1.6669x; 1.0635x over previous
import functools

import jax
import jax.numpy as jnp
from jax.experimental import pallas as pl
from jax.experimental.pallas import tpu as pltpu


def _pick(n, cands):
    for c in cands:
        if n % c == 0:
            return c
    return n


_DN = (((1,), (1,)), ((), ()))


def _main_body(s1_ref, s2_ref, w_ref, adj_ref, mskT_ref, bias_ref, a_ref,
               out_ref, acc_ref, fts_ref, *, Dh):
    i = pl.program_id(0)

    @pl.when(i == 0)
    def _():
        w = w_ref[:]
        fts_ref[:, :Dh] = jax.lax.dot_general(
            s1_ref[:], w, _DN, preferred_element_type=jnp.float32
        ).astype(jnp.bfloat16)
        fts_ref[:, Dh:] = jax.lax.dot_general(
            s2_ref[:], w, _DN, preferred_element_type=jnp.float32
        ).astype(jnp.bfloat16)

    h = jax.lax.dot_general(
        adj_ref[:].astype(jnp.bfloat16), fts_ref[:],
        (((1,), (0,)), ((), ())),
        preferred_element_type=jnp.float32)
    h = h + bias_ref[:]
    a = a_ref[0, 0]
    h = jnp.where(h >= 0, h, a * h)
    out_ref[:] = h.astype(jnp.bfloat16)
    part = jnp.sum(h[:, :Dh] * mskT_ref[:], axis=0, keepdims=True)

    @pl.when(i == 0)
    def _():
        acc_ref[:] = part

    @pl.when(i != 0)
    def _():
        acc_ref[:] = acc_ref[:] + part


def _epi_body(h_ref, hsum_ref, msk_ref, bw_ref, sb_ref, b_ref, out_ref,
              v_ref, *, Dh):
    i = pl.program_id(0)

    @pl.when(i == 0)
    def _():
        total = jnp.sum(msk_ref[:])
        c = jax.nn.sigmoid(hsum_ref[:] / total)
        v_ref[:] = jax.lax.dot_general(
            c, bw_ref[:], _DN, preferred_element_type=jnp.float32)

    v = v_ref[:]
    h = h_ref[:].astype(jnp.float32)
    b = b_ref[0, 0]
    sc1 = jnp.sum(h[:, :Dh] * v, axis=1, keepdims=True)
    sc2 = jnp.sum(h[:, Dh:] * v, axis=1, keepdims=True)
    out_ref[:] = jnp.concatenate([sc1, sc2], axis=1) + sb_ref[:] + b


def kernel(seq1, seq2, adj, sparse, msk, samp_bias1, samp_bias2,
           W, prelu_a, gcn_bias, bil_w, bil_b):
    B, N, Din = seq1.shape
    Dh = W.shape[0]
    s1 = seq1[0]
    s2 = seq2[0]
    A = adj[0]
    mskT = msk.reshape(N, 1)
    a11 = prelu_a.reshape(1, 1)
    b11 = bil_b.reshape(1, 1)
    bias_cat = jnp.concatenate([gcn_bias, gcn_bias]).reshape(1, 2 * Dh)
    bw = bil_w[0]
    sbcat = jnp.concatenate(
        [samp_bias1.reshape(N, 1), samp_bias2.reshape(N, 1)], axis=1)

    R = _pick(N, (400, 200, 8))
    h_cat, hsum = pl.pallas_call(
        functools.partial(_main_body, Dh=Dh),
        grid=(N // R,),
        in_specs=[
            pl.BlockSpec((N, Din), lambda i: (0, 0)),
            pl.BlockSpec((N, Din), lambda i: (0, 0)),
            pl.BlockSpec((Dh, Din), lambda i: (0, 0)),
            pl.BlockSpec((R, N), lambda i: (i, 0)),
            pl.BlockSpec((R, 1), lambda i: (i, 0)),
            pl.BlockSpec((1, 2 * Dh), lambda i: (0, 0)),
            pl.BlockSpec(memory_space=pltpu.SMEM),
        ],
        out_specs=[
            pl.BlockSpec((R, 2 * Dh), lambda i: (i, 0)),
            pl.BlockSpec((1, Dh), lambda i: (0, 0)),
        ],
        out_shape=[
            jax.ShapeDtypeStruct((N, 2 * Dh), jnp.bfloat16),
            jax.ShapeDtypeStruct((1, Dh), jnp.float32),
        ],
        scratch_shapes=[pltpu.VMEM((N, 2 * Dh), jnp.bfloat16)],
    )(s1, s2, W, A, mskT, bias_cat, a11)

    Rs = _pick(N, (2000, 1000, 500, 8))
    sc = pl.pallas_call(
        functools.partial(_epi_body, Dh=Dh),
        grid=(N // Rs,),
        in_specs=[
            pl.BlockSpec((Rs, 2 * Dh), lambda i: (i, 0)),
            pl.BlockSpec((1, Dh), lambda i: (0, 0)),
            pl.BlockSpec((1, N), lambda i: (0, 0)),
            pl.BlockSpec((Dh, Dh), lambda i: (0, 0)),
            pl.BlockSpec((Rs, 2), lambda i: (i, 0)),
            pl.BlockSpec(memory_space=pltpu.SMEM),
        ],
        out_specs=pl.BlockSpec((Rs, 2), lambda i: (i, 0)),
        out_shape=jax.ShapeDtypeStruct((N, 2), jnp.float32),
        scratch_shapes=[pltpu.VMEM((1, Dh), jnp.float32)],
    )(h_cat, hsum, msk, bw, sbcat, b11)

    return jnp.concatenate([sc[:, 0].reshape(1, N), sc[:, 1].reshape(1, N)],
                           axis=1)
